# baseline (speedup 1.0000x reference)
import functools

import jax
import jax.numpy as jnp
from jax import lax
from jax.experimental import pallas as pl
from jax.experimental.pallas import tpu as pltpu
from jax.experimental.pallas import tpu_sc as plsc

N = 100000
E = 1600000
D = 48
DE = 16
T = 128
C16 = 16


BN = 2000
NGRID = N // BN
BEB = 8000
EGRID = E // BEB


def _embed_body(w_ref, m_ref, o_ref):
    o_ref[...] = w_ref[...] * m_ref[...]


_embed = pl.pallas_call(
    _embed_body,
    out_shape=jax.ShapeDtypeStruct((T, D), jnp.float32),
)


def _pre_body(x_ref, wt_ref, b_ref, y0_ref, y1_ref, y2_ref):
    y = jnp.dot(x_ref[...], wt_ref[...], preferred_element_type=jnp.float32)
    y = y + b_ref[...]
    y0_ref[...] = y[:, 0:16]
    y1_ref[...] = y[:, 16:32]
    y2_ref[...] = y[:, 32:48]


_pre = pl.pallas_call(
    _pre_body,
    grid=(NGRID,),
    in_specs=[
        pl.BlockSpec((BN, D), lambda i: (i, 0)),
        pl.BlockSpec((D, D), lambda i: (0, 0)),
        pl.BlockSpec((1, D), lambda i: (0, 0)),
    ],
    out_specs=[
        pl.BlockSpec((BN, C16), lambda i: (i, 0)),
        pl.BlockSpec((BN, C16), lambda i: (i, 0)),
        pl.BlockSpec((BN, C16), lambda i: (i, 0)),
    ],
    out_shape=[jax.ShapeDtypeStruct((N, C16), jnp.float32)] * 3,
)


_E8 = E // 8
_BE8 = 8000
_EGRID8 = _E8 // _BE8


def _eproj_body(ea_ref, w0_ref, w1_ref, w2_ref, e0_ref, e1_ref, e2_ref):
    x = ea_ref[...]
    e0_ref[...] = jnp.dot(x, w0_ref[...], preferred_element_type=jnp.float32)
    e1_ref[...] = jnp.dot(x, w1_ref[...], preferred_element_type=jnp.float32)
    e2_ref[...] = jnp.dot(x, w2_ref[...], preferred_element_type=jnp.float32)


_eproj = pl.pallas_call(
    _eproj_body,
    grid=(_EGRID8,),
    in_specs=[
        pl.BlockSpec((_BE8, 128), lambda i: (i, 0)),
        pl.BlockSpec((128, 128), lambda i: (0, 0)),
        pl.BlockSpec((128, 128), lambda i: (0, 0)),
        pl.BlockSpec((128, 128), lambda i: (0, 0)),
    ],
    out_specs=[
        pl.BlockSpec((_BE8, 128), lambda i: (i, 0)),
        pl.BlockSpec((_BE8, 128), lambda i: (i, 0)),
        pl.BlockSpec((_BE8, 128), lambda i: (i, 0)),
    ],
    out_shape=[jax.ShapeDtypeStruct((_E8, 128), jnp.float32)] * 3,
)


def _upd_body(agg_ref, x_ref, ws_ref, wu_ref, z_ref, can_ref, o_ref):
    a = agg_ref[...]
    agg = jnp.concatenate(
        [a[0, 0] + a[1, 0], a[0, 1] + a[1, 1], a[0, 2] + a[1, 2]], axis=1)
    gate = 1.0 / (1.0 + jnp.exp(-z_ref[...]))
    h = jnp.dot(agg, wu_ref[...], preferred_element_type=jnp.float32)
    h = h + jnp.dot(x_ref[...], ws_ref[...], preferred_element_type=jnp.float32)
    o_ref[...] = jnp.maximum(h, 0.0) * gate + can_ref[...]


_upd = pl.pallas_call(
    _upd_body,
    grid=(NGRID,),
    in_specs=[
        pl.BlockSpec((2, 3, BN, C16), lambda i: (0, 0, i, 0)),
        pl.BlockSpec((BN, D), lambda i: (i, 0)),
        pl.BlockSpec((D, D), lambda i: (0, 0)),
        pl.BlockSpec((D, D), lambda i: (0, 0)),
        pl.BlockSpec((BN, 1), lambda i: (i, 0)),
        pl.BlockSpec((BN, 1), lambda i: (i, 0)),
    ],
    out_specs=pl.BlockSpec((BN, D), lambda i: (i, 0)),
    out_shape=jax.ShapeDtypeStruct((N, D), jnp.float32),
)


def _head_body(x_ref, wh_ref, bh_ref, o_ref):
    o_ref[...] = jnp.dot(x_ref[...], wh_ref[...],
                         preferred_element_type=jnp.float32) + bh_ref[...]


_head = pl.pallas_call(
    _head_body,
    grid=(NGRID,),
    in_specs=[
        pl.BlockSpec((BN, D), lambda i: (i, 0)),
        pl.BlockSpec((D, 1), lambda i: (0, 0)),
        pl.BlockSpec((1, 1), lambda i: (0, 0)),
    ],
    out_specs=pl.BlockSpec((BN, 1), lambda i: (i, 0)),
    out_shape=jax.ShapeDtypeStruct((N, 1), jnp.float32),
)


_mesh = plsc.VectorSubcoreMesh(core_axis_name="c", subcore_axis_name="s")

_BNODE = 2000
_NBLK_NODE = N // _BNODE


def _embed_gather_body(g_hbm, idx_hbm, out_hbm, idx_v, rows_v, sem):
    cid = lax.axis_index("c")
    sid = lax.axis_index("s")
    wid = cid * 16 + sid
    for j in range(2):
        blk = wid + 32 * j

        @pl.when(blk < _NBLK_NODE)
        def _():
            pltpu.sync_copy(idx_hbm.at[pl.ds(blk * _BNODE, _BNODE)], idx_v)
            pltpu.async_copy(g_hbm.at[idx_v], rows_v, sem).wait()
            pltpu.sync_copy(rows_v, out_hbm.at[pl.ds(blk * _BNODE, _BNODE)])


_embed_gather = pl.kernel(
    _embed_gather_body,
    out_type=jax.ShapeDtypeStruct((N, D), jnp.float32),
    mesh=_mesh,
    compiler_params=pltpu.CompilerParams(use_tc_tiling_on_sc=False),
    scratch_types=[
        pltpu.VMEM((_BNODE,), jnp.int32),
        pltpu.VMEM((_BNODE, D), jnp.float32),
        pltpu.SemaphoreType.DMA,
    ],
)

_BEDGE = 400
_NBLK_EDGE = E // _BEDGE
_ITER_EDGE = _NBLK_EDGE // 32
_NPS = N // 16


def _edge_body(y0, y1, y2, e0, e1, e2, src_hbm, dst_hbm, zero_hbm, out_hbm,
               src_v, dst_v, rows_v, ep_v, agg_sh, lsem, gsem, ssem):
    cid = lax.axis_index("c")
    sid = lax.axis_index("s")
    wid = cid * 16 + sid
    ys = (y0, y1, y2)
    es = (e0, e1, e2)

    for c in range(3):
        yc = ys[c]
        ec = es[c]

        def issue_loads(t, b):
            base = (wid + 32 * t) * _BEDGE
            pltpu.async_copy(src_hbm.at[pl.ds(base, _BEDGE)], src_v.at[b],
                             lsem[b])
            pltpu.async_copy(dst_hbm.at[pl.ds(base, _BEDGE)], dst_v.at[b],
                             lsem[b])
            pltpu.async_copy(ec.at[pl.ds(base, _BEDGE)], ep_v.at[b], lsem[b])

        def wait_loads(t, b):
            base = (wid + 32 * t) * _BEDGE
            pltpu.make_async_copy(src_hbm.at[pl.ds(base, _BEDGE)],
                                  src_v.at[b], lsem[b]).wait()
            pltpu.make_async_copy(dst_hbm.at[pl.ds(base, _BEDGE)],
                                  dst_v.at[b], lsem[b]).wait()
            pltpu.make_async_copy(ec.at[pl.ds(base, _BEDGE)], ep_v.at[b],
                                  lsem[b]).wait()

        def wait_scatter(b):
            pltpu.make_async_copy(rows_v.at[b], agg_sh.at[dst_v.at[b]],
                                  ssem[b]).wait()

        def one_iter(t, x, is_tail):
            wait_loads(t, x)
            g = pltpu.async_copy(yc.at[src_v.at[x]], rows_v.at[x], gsem[x])
            y = (x + 1) % 2

            if not is_tail:
                @pl.when(t >= 1)
                def _():
                    wait_scatter(y)

                issue_loads(t + 1, y)

            g.wait()

            def rbody(r, carry2):
                for u in range(10):
                    bb = r * 10 + u
                    v = rows_v[x, bb] + ep_v[x, bb]
                    rows_v[x, bb] = jnp.maximum(v, 0.0)
                return 0

            lax.fori_loop(0, _BEDGE // 10, rbody, 0)
            pltpu.async_copy(rows_v.at[x], agg_sh.at[dst_v.at[x]], ssem[x],
                             add=True)

        pltpu.sync_copy(zero_hbm.at[pl.ds(sid * _NPS, _NPS)],
                        agg_sh.at[pl.ds(sid * _NPS, _NPS)])
        plsc.subcore_barrier()

        issue_loads(0, 0)

        def pair_body(p, carry):
            t0 = 2 * p
            one_iter(t0, 0, False)
            one_iter(t0 + 1, 1, False)
            return 0

        lax.fori_loop(0, (_ITER_EDGE - 1) // 2, pair_body, 0)
        one_iter(_ITER_EDGE - 1, (_ITER_EDGE - 1) % 2, True)
        wait_scatter((_ITER_EDGE - 2) % 2)
        wait_scatter((_ITER_EDGE - 1) % 2)

        plsc.subcore_barrier()
        base = (cid * 3 + c) * N + sid * _NPS
        pltpu.sync_copy(agg_sh.at[pl.ds(sid * _NPS, _NPS)],
                        out_hbm.at[pl.ds(base, _NPS)])
        plsc.subcore_barrier()


_edge = pl.kernel(
    _edge_body,
    out_type=jax.ShapeDtypeStruct((6 * N, C16), jnp.float32),
    mesh=_mesh,
    compiler_params=pltpu.CompilerParams(use_tc_tiling_on_sc=False),
    scratch_types=[
        pltpu.VMEM((2, _BEDGE), jnp.int32),
        pltpu.VMEM((2, _BEDGE), jnp.int32),
        pltpu.VMEM((2, _BEDGE, C16), jnp.float32),
        pltpu.VMEM((2, _BEDGE, C16), jnp.float32),
        pltpu.VMEM_SHARED((N, C16), jnp.float32),
        [pltpu.SemaphoreType.DMA] * 2,
        [pltpu.SemaphoreType.DMA] * 2,
        [pltpu.SemaphoreType.DMA] * 2,
    ],
)


def kernel(node_idx, edge_index, edge_attr, z, canonical, W_embed, mask,
           W_msg, b_msg, W_self, W_upd, W_head, b_head):
    f32 = jnp.float32
    src = edge_index[0]
    dst = edge_index[1]
    idx = node_idx.astype(jnp.int32)
    zeros16 = jnp.zeros((N, C16), f32)
    z2 = z.reshape(N, 1)
    can2 = canonical.reshape(N, 1)

    G = _embed(W_embed, mask)
    x = _embed_gather(G, idx)
    ea8 = edge_attr.reshape(_E8, 128)
    eye8 = jnp.eye(8, dtype=f32)

    for l in range(4):
        wt = W_msg[l, :D, :]
        wb = W_msg[l, D:, :]
        bl = b_msg[l].reshape(1, D)
        y0, y1, y2 = _pre(x, wt, bl)
        w8 = [jnp.kron(eye8, wb[:, 16 * c:16 * c + 16]) for c in range(3)]
        ep0, ep1, ep2 = _eproj(ea8, w8[0], w8[1], w8[2])
        aggp = _edge(y0, y1, y2,
                     ep0.reshape(E, C16), ep1.reshape(E, C16),
                     ep2.reshape(E, C16), src, dst, zeros16)
        aggp = aggp.reshape(2, 3, N, C16)
        x = _upd(aggp, x, W_self[l], W_upd[l], z2, can2)

    return _head(x, W_head, b_head.reshape(1, 1))


# speedup vs baseline: 1.0935x; 1.0935x over previous
import functools

import jax
import jax.numpy as jnp
from jax import lax
from jax.experimental import pallas as pl
from jax.experimental.pallas import tpu as pltpu
from jax.experimental.pallas import tpu_sc as plsc

N = 100000
E = 1600000
D = 48
DE = 16
T = 128
C16 = 16


BN = 2000
NGRID = N // BN
BEB = 8000
EGRID = E // BEB


def _embed_body(w_ref, m_ref, o_ref):
    o_ref[...] = w_ref[...] * m_ref[...]


_embed = pl.pallas_call(
    _embed_body,
    out_shape=jax.ShapeDtypeStruct((T, D), jnp.float32),
)


_N8 = N // 8
_BN8 = _N8
_NGRID8 = 1


def _pre_body(x_ref, w0_ref, w1_ref, w2_ref, b_ref, y0_ref, y1_ref, y2_ref):
    x = x_ref[...]
    b = b_ref[...]
    y0_ref[...] = jnp.dot(x, w0_ref[...],
                          preferred_element_type=jnp.float32) + b[0:1, :]
    y1_ref[...] = jnp.dot(x, w1_ref[...],
                          preferred_element_type=jnp.float32) + b[1:2, :]
    y2_ref[...] = jnp.dot(x, w2_ref[...],
                          preferred_element_type=jnp.float32) + b[2:3, :]


_pre = pl.pallas_call(
    _pre_body,
    grid=(_NGRID8,),
    in_specs=[
        pl.BlockSpec((_BN8, 384), lambda i: (i, 0)),
        pl.BlockSpec((384, 128), lambda i: (0, 0)),
        pl.BlockSpec((384, 128), lambda i: (0, 0)),
        pl.BlockSpec((384, 128), lambda i: (0, 0)),
        pl.BlockSpec((3, 128), lambda i: (0, 0)),
    ],
    out_specs=[
        pl.BlockSpec((_BN8, 128), lambda i: (i, 0)),
        pl.BlockSpec((_BN8, 128), lambda i: (i, 0)),
        pl.BlockSpec((_BN8, 128), lambda i: (i, 0)),
    ],
    out_shape=[jax.ShapeDtypeStruct((_N8, 128), jnp.float32)] * 3,
)


_E8 = E // 8
_BE8 = 8000
_EGRID8 = _E8 // _BE8


def _eproj_body(ea_ref, w0_ref, w1_ref, w2_ref, e0_ref, e1_ref, e2_ref):
    x = ea_ref[...]
    e0_ref[...] = jnp.dot(x, w0_ref[...], preferred_element_type=jnp.float32)
    e1_ref[...] = jnp.dot(x, w1_ref[...], preferred_element_type=jnp.float32)
    e2_ref[...] = jnp.dot(x, w2_ref[...], preferred_element_type=jnp.float32)


_eproj = pl.pallas_call(
    _eproj_body,
    grid=(_EGRID8,),
    in_specs=[
        pl.BlockSpec((_BE8, 128), lambda i: (i, 0)),
        pl.BlockSpec((128, 128), lambda i: (0, 0)),
        pl.BlockSpec((128, 128), lambda i: (0, 0)),
        pl.BlockSpec((128, 128), lambda i: (0, 0)),
    ],
    out_specs=[
        pl.BlockSpec((_BE8, 128), lambda i: (i, 0)),
        pl.BlockSpec((_BE8, 128), lambda i: (i, 0)),
        pl.BlockSpec((_BE8, 128), lambda i: (i, 0)),
    ],
    out_shape=[jax.ShapeDtypeStruct((_E8, 128), jnp.float32)] * 3,
)


def _upd_body(agg_ref, x_ref, ws_ref, wu_ref, z_ref, can_ref, o_ref):
    a = agg_ref[...]
    agg = jnp.concatenate(
        [a[0, 0] + a[1, 0], a[0, 1] + a[1, 1], a[0, 2] + a[1, 2]], axis=1)
    gate = 1.0 / (1.0 + jnp.exp(-z_ref[...]))
    h = jnp.dot(agg, wu_ref[...], preferred_element_type=jnp.float32)
    h = h + jnp.dot(x_ref[...], ws_ref[...], preferred_element_type=jnp.float32)
    o_ref[...] = jnp.maximum(h, 0.0) * gate + can_ref[...]


_upd = pl.pallas_call(
    _upd_body,
    grid=(NGRID,),
    in_specs=[
        pl.BlockSpec((2, 3, BN, C16), lambda i: (0, 0, i, 0)),
        pl.BlockSpec((BN, D), lambda i: (i, 0)),
        pl.BlockSpec((D, D), lambda i: (0, 0)),
        pl.BlockSpec((D, D), lambda i: (0, 0)),
        pl.BlockSpec((BN, 1), lambda i: (i, 0)),
        pl.BlockSpec((BN, 1), lambda i: (i, 0)),
    ],
    out_specs=pl.BlockSpec((BN, D), lambda i: (i, 0)),
    out_shape=jax.ShapeDtypeStruct((N, D), jnp.float32),
)


def _head_body(x_ref, wh_ref, bh_ref, o_ref):
    o_ref[...] = jnp.dot(x_ref[...], wh_ref[...],
                         preferred_element_type=jnp.float32) + bh_ref[...]


_head = pl.pallas_call(
    _head_body,
    grid=(NGRID,),
    in_specs=[
        pl.BlockSpec((BN, D), lambda i: (i, 0)),
        pl.BlockSpec((D, 1), lambda i: (0, 0)),
        pl.BlockSpec((1, 1), lambda i: (0, 0)),
    ],
    out_specs=pl.BlockSpec((BN, 1), lambda i: (i, 0)),
    out_shape=jax.ShapeDtypeStruct((N, 1), jnp.float32),
)


_mesh = plsc.VectorSubcoreMesh(core_axis_name="c", subcore_axis_name="s")

_BNODE = 2000
_NBLK_NODE = N // _BNODE


def _embed_gather_body(g_hbm, idx_hbm, out_hbm, idx_v, rows_v, sem):
    cid = lax.axis_index("c")
    sid = lax.axis_index("s")
    wid = cid * 16 + sid
    for j in range(2):
        blk = wid + 32 * j

        @pl.when(blk < _NBLK_NODE)
        def _():
            pltpu.sync_copy(idx_hbm.at[pl.ds(blk * _BNODE, _BNODE)], idx_v)
            pltpu.async_copy(g_hbm.at[idx_v], rows_v, sem).wait()
            pltpu.sync_copy(rows_v, out_hbm.at[pl.ds(blk * _BNODE, _BNODE)])


_embed_gather = pl.kernel(
    _embed_gather_body,
    out_type=jax.ShapeDtypeStruct((N, D), jnp.float32),
    mesh=_mesh,
    compiler_params=pltpu.CompilerParams(use_tc_tiling_on_sc=False),
    scratch_types=[
        pltpu.VMEM((_BNODE,), jnp.int32),
        pltpu.VMEM((_BNODE, D), jnp.float32),
        pltpu.SemaphoreType.DMA,
    ],
)

_BEDGE = 400
_NBLK_EDGE = E // _BEDGE
_ITER_EDGE = _NBLK_EDGE // 32
_NPS = N // 16


def _edge_body(y0, y1, y2, e0, e1, e2, src_hbm, dst_hbm, zero_hbm, out_hbm,
               src_v, dst_v, rows_v, ep_v, agg_sh, lsem, gsem, ssem):
    cid = lax.axis_index("c")
    sid = lax.axis_index("s")
    wid = cid * 16 + sid
    ys = (y0, y1, y2)
    es = (e0, e1, e2)

    for c in range(3):
        yc = ys[c]
        ec = es[c]

        def issue_loads(t, b):
            base = (wid + 32 * t) * _BEDGE
            pltpu.async_copy(src_hbm.at[pl.ds(base, _BEDGE)], src_v.at[b],
                             lsem[b])
            pltpu.async_copy(dst_hbm.at[pl.ds(base, _BEDGE)], dst_v.at[b],
                             lsem[b])
            pltpu.async_copy(ec.at[pl.ds(base, _BEDGE)], ep_v.at[b], lsem[b])

        def wait_loads(t, b):
            base = (wid + 32 * t) * _BEDGE
            pltpu.make_async_copy(src_hbm.at[pl.ds(base, _BEDGE)],
                                  src_v.at[b], lsem[b]).wait()
            pltpu.make_async_copy(dst_hbm.at[pl.ds(base, _BEDGE)],
                                  dst_v.at[b], lsem[b]).wait()
            pltpu.make_async_copy(ec.at[pl.ds(base, _BEDGE)], ep_v.at[b],
                                  lsem[b]).wait()

        def wait_scatter(b):
            pltpu.make_async_copy(rows_v.at[b], agg_sh.at[dst_v.at[b]],
                                  ssem[b]).wait()

        def one_iter(t, x, is_tail):
            wait_loads(t, x)
            g = pltpu.async_copy(yc.at[src_v.at[x]], rows_v.at[x], gsem[x])
            y = (x + 1) % 2

            if not is_tail:
                @pl.when(t >= 1)
                def _():
                    wait_scatter(y)

                issue_loads(t + 1, y)

            g.wait()

            def rbody(r, carry2):
                for u in range(10):
                    bb = r * 10 + u
                    v = rows_v[x, bb] + ep_v[x, bb]
                    rows_v[x, bb] = jnp.maximum(v, 0.0)
                return 0

            lax.fori_loop(0, _BEDGE // 10, rbody, 0)
            pltpu.async_copy(rows_v.at[x], agg_sh.at[dst_v.at[x]], ssem[x],
                             add=True)

        pltpu.sync_copy(zero_hbm.at[pl.ds(sid * _NPS, _NPS)],
                        agg_sh.at[pl.ds(sid * _NPS, _NPS)])
        plsc.subcore_barrier()

        issue_loads(0, 0)

        def pair_body(p, carry):
            t0 = 2 * p
            one_iter(t0, 0, False)
            one_iter(t0 + 1, 1, False)
            return 0

        lax.fori_loop(0, (_ITER_EDGE - 1) // 2, pair_body, 0)
        one_iter(_ITER_EDGE - 1, (_ITER_EDGE - 1) % 2, True)
        wait_scatter((_ITER_EDGE - 2) % 2)
        wait_scatter((_ITER_EDGE - 1) % 2)

        plsc.subcore_barrier()
        base = (cid * 3 + c) * N + sid * _NPS
        pltpu.sync_copy(agg_sh.at[pl.ds(sid * _NPS, _NPS)],
                        out_hbm.at[pl.ds(base, _NPS)])
        plsc.subcore_barrier()


_edge = pl.kernel(
    _edge_body,
    out_type=jax.ShapeDtypeStruct((6 * N, C16), jnp.float32),
    mesh=_mesh,
    compiler_params=pltpu.CompilerParams(use_tc_tiling_on_sc=False),
    scratch_types=[
        pltpu.VMEM((2, _BEDGE), jnp.int32),
        pltpu.VMEM((2, _BEDGE), jnp.int32),
        pltpu.VMEM((2, _BEDGE, C16), jnp.float32),
        pltpu.VMEM((2, _BEDGE, C16), jnp.float32),
        pltpu.VMEM_SHARED((N, C16), jnp.float32),
        [pltpu.SemaphoreType.DMA] * 2,
        [pltpu.SemaphoreType.DMA] * 2,
        [pltpu.SemaphoreType.DMA] * 2,
    ],
)


def kernel(node_idx, edge_index, edge_attr, z, canonical, W_embed, mask,
           W_msg, b_msg, W_self, W_upd, W_head, b_head):
    f32 = jnp.float32
    src = edge_index[0]
    dst = edge_index[1]
    idx = node_idx.astype(jnp.int32)
    zeros16 = jnp.zeros((N, C16), f32)
    z2 = z.reshape(N, 1)
    can2 = canonical.reshape(N, 1)

    G = _embed(W_embed, mask)
    x = _embed_gather(G, idx)
    ea8 = edge_attr.reshape(_E8, 128)
    eye8 = jnp.eye(8, dtype=f32)

    for l in range(4):
        wt = W_msg[l, :D, :]
        wb = W_msg[l, D:, :]
        bl = b_msg[l].reshape(1, D)
        wt8 = [jnp.kron(eye8, wt[:, 16 * c:16 * c + 16]) for c in range(3)]
        bl8 = jnp.stack([jnp.tile(bl[0, 16 * c:16 * c + 16], 8)
                         for c in range(3)])
        y0, y1, y2 = _pre(x.reshape(_N8, 384), wt8[0], wt8[1], wt8[2], bl8)
        w8 = [jnp.kron(eye8, wb[:, 16 * c:16 * c + 16]) for c in range(3)]
        ep0, ep1, ep2 = _eproj(ea8, w8[0], w8[1], w8[2])
        aggp = _edge(y0.reshape(N, C16), y1.reshape(N, C16),
                     y2.reshape(N, C16),
                     ep0.reshape(E, C16), ep1.reshape(E, C16),
                     ep2.reshape(E, C16), src, dst, zeros16)
        aggp = aggp.reshape(2, 3, N, C16)
        x = _upd(aggp, x, W_self[l], W_upd[l], z2, can2)

    return _head(x, W_head, b_head.reshape(1, 1))
